# baseline (speedup 1.0000x reference)
import functools

import numpy as np
import jax
import jax.numpy as jnp
from jax import lax
from jax.experimental import pallas as pl
from jax.experimental.pallas import tpu as pltpu
from jax.experimental.pallas import tpu_sc as plsc

_DN_T = (((1,), (1,)), ((), ()))


def _taps(xfn, w_ref, sb_ref):
    acc = None
    for p in (0, 1):
        for q in (0, 1):
            part = lax.dot_general(xfn(p, q), w_ref[2 * p + q], _DN_T,
                                   preferred_element_type=jnp.float32)
            acc = part if acc is None else acc + part
    acc = acc * sb_ref[0:1, :] + sb_ref[1:2, :]
    return jnp.where(acc >= 0.0, acc, np.float32(0.2) * acc)


def _ref_tap(x_ref):
    G, U, _, W2, D = x_ref.shape
    V = W2 // 2
    M = G * U * V
    planes = [x_ref[:, :, p] for p in (0, 1)]

    def xfn(p, q):
        return planes[p].reshape(G, U, V, 2, D)[:, :, :, q, :].reshape(M, D)

    return xfn


def _val_tap(y, U, V, D):
    def xfn(p, q):
        xp = y.reshape(U, 2, 2 * V, D)[:, p]
        return xp.reshape(U, V, 2, D)[:, :, q, :].reshape(U * V, D)

    return xfn


def _conv1_body(x_ref, w_ref, sb_ref, o_ref):
    o_ref[...] = _taps(_ref_tap(x_ref), w_ref, sb_ref)


def _conv1_call(z5, wst, sb, nsteps):
    N, U, _, W2, D = z5.shape
    V = W2 // 2
    g = N // nsteps
    mb = g * U * V
    return pl.pallas_call(
        _conv1_body,
        grid=(nsteps,),
        in_specs=[
            pl.BlockSpec((g, U, 2, W2, D), lambda j: (j, 0, 0, 0, 0)),
            pl.BlockSpec((4, D, D), lambda j: (0, 0, 0)),
            pl.BlockSpec((2, D), lambda j: (0, 0)),
        ],
        out_specs=pl.BlockSpec((mb, D), lambda j: (j, 0)),
        out_shape=jax.ShapeDtypeStruct((N * U * V, D), jnp.float32),
    )(z5, wst, sb)


def _conv2_body(x_ref, w_ref, sb_ref, o_ref):
    y = _taps(_ref_tap(x_ref), w_ref, sb_ref)
    D = y.shape[-1]
    y5 = y.reshape(2, 8, 8, 8, D)
    o_ref[...] = jnp.transpose(y5, (0, 2, 1, 3, 4)).reshape(16, 64, D)


def _conv2_call(z5, wst, sb):
    N, U, _, W2, D = z5.shape
    return pl.pallas_call(
        _conv2_body,
        grid=(4,),
        in_specs=[
            pl.BlockSpec((16, U, 2, W2, D), lambda j: (j, 0, 0, 0, 0)),
            pl.BlockSpec((4, D, D), lambda j: (0, 0, 0)),
            pl.BlockSpec((2, D), lambda j: (0, 0)),
        ],
        out_specs=pl.BlockSpec((16, 64, D), lambda j: (j, 0, 0)),
        out_shape=jax.ShapeDtypeStruct((64, 64, D), jnp.float32),
    )(z5, wst, sb)


def _tail_body(m_ref, w3_ref, sb3_ref, w4_ref, sb4_ref, w5_ref, sb5_ref,
               fw_ref, fb_ref, o_ref):
    D = m_ref.shape[-1]
    y3 = _taps(_ref_tap(m_ref), w3_ref, sb3_ref)
    y4 = _taps(_val_tap(y3, 16, 16, D), w4_ref, sb4_ref)
    y5 = _taps(_val_tap(y4, 8, 8, D), w5_ref, sb5_ref)
    h = jnp.sum(y5 * fw_ref[...], axis=1, keepdims=True) + fb_ref[0:1, 0:1]
    o_ref[...] = h


def _tail_call(m3, w3, sb3, w4, sb4, w5, sb5, fw, fb):
    return pl.pallas_call(
        _tail_body,
        out_shape=jax.ShapeDtypeStruct((64, 1), jnp.float32),
    )(m3, w3, sb3, w4, sb4, w5, sb5, fw, fb)


def _make_gate_body(E):
    def _gate_body(h_ref, c_ref, s_ref, wg_ref, o_ref):
        h = h_ref[...]
        re = jnp.dot(h, c_ref[...], preferred_element_type=jnp.float32)
        im = jnp.dot(h, s_ref[...], preferred_element_type=jnp.float32)
        amp = jnp.sqrt(re * re + im * im)
        lg = jnp.dot(amp, wg_ref[...], preferred_element_type=jnp.float32)
        col = lax.broadcasted_iota(jnp.int32, lg.shape, 1)
        o_ref[...] = jnp.where(col < E, lg, np.float32(-1e30))

    return _gate_body


def _logits_call(h4, cm, sm, wgp, E):
    B = h4.shape[0]
    return pl.pallas_call(
        _make_gate_body(E),
        out_shape=jax.ShapeDtypeStruct((B, 128), jnp.float32),
    )(h4, cm, sm, wgp)


def _sc_gate(logits_p):
    B = logits_p.shape[0]
    mesh = plsc.VectorSubcoreMesh(core_axis_name="c", subcore_axis_name="s")

    @functools.partial(
        pl.kernel,
        out_type=jax.ShapeDtypeStruct((B, 16), jnp.float32),
        mesh=mesh,
        compiler_params=pltpu.CompilerParams(needs_layout_passes=False),
        scratch_types=[
            pltpu.VMEM((16,), jnp.float32),
            pltpu.VMEM((16,), jnp.float32),
            pltpu.VMEM((16,), jnp.float32),
        ],
    )
    def k(lg_hbm, out_hbm, lvec, svec, gvec):
        wid = lax.axis_index("s") * 2 + lax.axis_index("c")

        @pl.when(wid < B)
        def _():
            pltpu.sync_copy(lg_hbm.at[wid, pl.ds(0, 16)], lvec)
            keys = lvec[...]
            lanes = lax.iota(jnp.int32, 16)
            sk, sv = plsc.sort_key_val(keys, lanes, descending=True)
            svec[...] = sk
            top1 = plsc.load_gather(svec, [jnp.zeros((16,), jnp.int32)])
            top2 = plsc.load_gather(svec, [jnp.full((16,), 1, jnp.int32)])
            e = jnp.exp(top2 - top1)
            one = jnp.full((16,), 1.0, jnp.float32)
            den = one + e
            g1 = one / den
            g2 = e / den
            vals = jnp.where(lanes == 0, g1, g2)
            gvec[...] = jnp.zeros((16,), jnp.float32)
            plsc.store_scatter(gvec, [sv], vals, mask=lanes < 2)
            pltpu.sync_copy(gvec, out_hbm.at[wid])

    return k(logits_p)


def kernel(x, conv_ws, conv_bs, bn_gammas, bn_betas, fuse_w, fuse_b, w_gate,
           training):
    B, T, H, W, D = x.shape
    N = B * T
    nf, E = w_gate.shape

    wsts, sbs = [], []
    for cw, cb, g, bb in zip(conv_ws, conv_bs, bn_gammas, bn_betas):
        wsts.append(jnp.transpose(cw, (2, 3, 0, 1)).reshape(4, D, D))
        sbs.append(jnp.stack([g, cb * g + bb]))

    z1 = x.reshape(N, H // 2, 2, W, D)
    y1 = _conv1_call(z1, wsts[0], sbs[0], 16)
    z2 = y1.reshape(N, 8, 2, 16, D)
    m3 = _conv2_call(z2, wsts[1], sbs[1])
    h = _tail_call(
        m3.reshape(1, 32, 2, 64, D), wsts[2], sbs[2], wsts[3], sbs[3],
        wsts[4], sbs[4], fuse_w.reshape(1, D), fuse_b.reshape(1, 1),
    )
    h4 = h.reshape(B, T)

    t = np.arange(T)[:, None].astype(np.float64)
    kk = np.arange(1, nf + 1)[None, :].astype(np.float64)
    ang = 2.0 * np.pi * t * kk / T
    cm = jnp.asarray(np.cos(ang) / np.sqrt(T), jnp.float32)
    sm = jnp.asarray(-np.sin(ang) / np.sqrt(T), jnp.float32)
    wgp = jnp.zeros((nf, 128), jnp.float32).at[:, :E].set(w_gate)

    lgp = _logits_call(h4, cm, sm, wgp, E)
    gates16 = _sc_gate(lgp)
    return gates16[:, :E]


# speedup vs baseline: 1.1701x; 1.1701x over previous
import functools

import numpy as np
import jax
import jax.numpy as jnp
from jax import lax
from jax.experimental import pallas as pl
from jax.experimental.pallas import tpu as pltpu
from jax.experimental.pallas import tpu_sc as plsc

_DN_T = (((1,), (1,)), ((), ()))


def _taps(xfn, w_ref, sb_ref):
    acc = None
    for p in (0, 1):
        for q in (0, 1):
            part = lax.dot_general(xfn(p, q), w_ref[2 * p + q], _DN_T,
                                   preferred_element_type=jnp.float32)
            acc = part if acc is None else acc + part
    acc = acc * sb_ref[0:1, :] + sb_ref[1:2, :]
    return jnp.where(acc >= 0.0, acc, np.float32(0.2) * acc)


def _ref_tap(x_ref):
    G, U, _, W2, D = x_ref.shape
    V = W2 // 2
    M = G * U * V
    planes = [x_ref[:, :, p] for p in (0, 1)]

    merged = [planes[p].reshape(M, 2 * D) for p in (0, 1)]

    def xfn(p, q):
        return merged[p][:, q * D:(q + 1) * D]

    return xfn


def _val_tap(y, U, V, D):
    def xfn(p, q):
        xp = y.reshape(U, 2, 2 * V, D)[:, p]
        return xp.reshape(U, V, 2, D)[:, :, q, :].reshape(U * V, D)

    return xfn


def _conv1_body(x_ref, w_ref, sb_ref, o_ref):
    o_ref[...] = _taps(_ref_tap(x_ref), w_ref, sb_ref)


def _conv1_call(z5, wst, sb, nsteps):
    N, U, _, W2, D = z5.shape
    V = W2 // 2
    g = N // nsteps
    mb = g * U * V
    return pl.pallas_call(
        _conv1_body,
        grid=(nsteps,),
        in_specs=[
            pl.BlockSpec((g, U, 2, W2, D), lambda j: (j, 0, 0, 0, 0)),
            pl.BlockSpec((4, D, D), lambda j: (0, 0, 0)),
            pl.BlockSpec((2, D), lambda j: (0, 0)),
        ],
        out_specs=pl.BlockSpec((mb, D), lambda j: (j, 0)),
        out_shape=jax.ShapeDtypeStruct((N * U * V, D), jnp.float32),
    )(z5, wst, sb)


def _conv2_body(x_ref, w_ref, sb_ref, o_ref):
    y = _taps(_ref_tap(x_ref), w_ref, sb_ref)
    D = y.shape[-1]
    y5 = y.reshape(2, 8, 8, 8, D)
    o_ref[...] = jnp.transpose(y5, (0, 2, 1, 3, 4)).reshape(16, 64, D)


def _conv2_call(z5, wst, sb):
    N, U, _, W2, D = z5.shape
    return pl.pallas_call(
        _conv2_body,
        grid=(4,),
        in_specs=[
            pl.BlockSpec((16, U, 2, W2, D), lambda j: (j, 0, 0, 0, 0)),
            pl.BlockSpec((4, D, D), lambda j: (0, 0, 0)),
            pl.BlockSpec((2, D), lambda j: (0, 0)),
        ],
        out_specs=pl.BlockSpec((16, 64, D), lambda j: (j, 0, 0)),
        out_shape=jax.ShapeDtypeStruct((64, 64, D), jnp.float32),
    )(z5, wst, sb)


def _tail_body(m_ref, w3_ref, sb3_ref, w4_ref, sb4_ref, w5_ref, sb5_ref,
               fw_ref, fb_ref, o_ref):
    D = m_ref.shape[-1]
    y3 = _taps(_ref_tap(m_ref), w3_ref, sb3_ref)
    y4 = _taps(_val_tap(y3, 16, 16, D), w4_ref, sb4_ref)
    y5 = _taps(_val_tap(y4, 8, 8, D), w5_ref, sb5_ref)
    h = jnp.sum(y5 * fw_ref[...], axis=1, keepdims=True) + fb_ref[0:1, 0:1]
    o_ref[...] = h


def _tail_call(m3, w3, sb3, w4, sb4, w5, sb5, fw, fb):
    return pl.pallas_call(
        _tail_body,
        out_shape=jax.ShapeDtypeStruct((64, 1), jnp.float32),
    )(m3, w3, sb3, w4, sb4, w5, sb5, fw, fb)


def _make_gate_body(E):
    def _gate_body(h_ref, c_ref, s_ref, wg_ref, o_ref):
        h = h_ref[...]
        re = jnp.dot(h, c_ref[...], preferred_element_type=jnp.float32)
        im = jnp.dot(h, s_ref[...], preferred_element_type=jnp.float32)
        amp = jnp.sqrt(re * re + im * im)
        lg = jnp.dot(amp, wg_ref[...], preferred_element_type=jnp.float32)
        col = lax.broadcasted_iota(jnp.int32, lg.shape, 1)
        o_ref[...] = jnp.where(col < E, lg, np.float32(-1e30))

    return _gate_body


def _logits_call(h4, cm, sm, wgp, E):
    B = h4.shape[0]
    return pl.pallas_call(
        _make_gate_body(E),
        out_shape=jax.ShapeDtypeStruct((B, 128), jnp.float32),
    )(h4, cm, sm, wgp)


def _sc_gate(logits_p):
    B = logits_p.shape[0]
    mesh = plsc.VectorSubcoreMesh(core_axis_name="c", subcore_axis_name="s")

    @functools.partial(
        pl.kernel,
        out_type=jax.ShapeDtypeStruct((B, 16), jnp.float32),
        mesh=mesh,
        compiler_params=pltpu.CompilerParams(needs_layout_passes=False),
        scratch_types=[
            pltpu.VMEM((16,), jnp.float32),
            pltpu.VMEM((16,), jnp.float32),
            pltpu.VMEM((16,), jnp.float32),
        ],
    )
    def k(lg_hbm, out_hbm, lvec, svec, gvec):
        wid = lax.axis_index("s") * 2 + lax.axis_index("c")

        @pl.when(wid < B)
        def _():
            pltpu.sync_copy(lg_hbm.at[wid, pl.ds(0, 16)], lvec)
            keys = lvec[...]
            lanes = lax.iota(jnp.int32, 16)
            sk, sv = plsc.sort_key_val(keys, lanes, descending=True)
            svec[...] = sk
            top1 = plsc.load_gather(svec, [jnp.zeros((16,), jnp.int32)])
            top2 = plsc.load_gather(svec, [jnp.full((16,), 1, jnp.int32)])
            e = jnp.exp(top2 - top1)
            one = jnp.full((16,), 1.0, jnp.float32)
            den = one + e
            g1 = one / den
            g2 = e / den
            vals = jnp.where(lanes == 0, g1, g2)
            gvec[...] = jnp.zeros((16,), jnp.float32)
            plsc.store_scatter(gvec, [sv], vals, mask=lanes < 2)
            pltpu.sync_copy(gvec, out_hbm.at[wid])

    return k(logits_p)


def kernel(x, conv_ws, conv_bs, bn_gammas, bn_betas, fuse_w, fuse_b, w_gate,
           training):
    B, T, H, W, D = x.shape
    N = B * T
    nf, E = w_gate.shape

    wsts, sbs = [], []
    for cw, cb, g, bb in zip(conv_ws, conv_bs, bn_gammas, bn_betas):
        wsts.append(jnp.transpose(cw, (2, 3, 0, 1)).reshape(4, D, D))
        sbs.append(jnp.stack([g, cb * g + bb]))

    z1 = x.reshape(N, H // 2, 2, W, D)
    y1 = _conv1_call(z1, wsts[0], sbs[0], 16)
    z2 = y1.reshape(N, 8, 2, 16, D)
    m3 = _conv2_call(z2, wsts[1], sbs[1])
    h = _tail_call(
        m3.reshape(1, 32, 2, 64, D), wsts[2], sbs[2], wsts[3], sbs[3],
        wsts[4], sbs[4], fuse_w.reshape(1, D), fuse_b.reshape(1, 1),
    )
    h4 = h.reshape(B, T)

    t = np.arange(T)[:, None].astype(np.float64)
    kk = np.arange(1, nf + 1)[None, :].astype(np.float64)
    ang = 2.0 * np.pi * t * kk / T
    cm = jnp.asarray(np.cos(ang) / np.sqrt(T), jnp.float32)
    sm = jnp.asarray(-np.sin(ang) / np.sqrt(T), jnp.float32)
    wgp = jnp.zeros((nf, 128), jnp.float32).at[:, :E].set(w_gate)

    lgp = _logits_call(h4, cm, sm, wgp, E)
    gates16 = _sc_gate(lgp)
    return gates16[:, :E]


# speedup vs baseline: 1.1711x; 1.0009x over previous
import functools

import numpy as np
import jax
import jax.numpy as jnp
from jax import lax
from jax.experimental import pallas as pl
from jax.experimental.pallas import tpu as pltpu
from jax.experimental.pallas import tpu_sc as plsc

_DN_T = (((1,), (1,)), ((), ()))


def _taps(xfn, w_ref, sb_ref):
    acc = None
    for p in (0, 1):
        for q in (0, 1):
            part = lax.dot_general(xfn(p, q), w_ref[2 * p + q], _DN_T,
                                   preferred_element_type=jnp.float32)
            acc = part if acc is None else acc + part
    acc = acc * sb_ref[0:1, :] + sb_ref[1:2, :]
    return jnp.where(acc >= 0.0, acc, np.float32(0.2) * acc)


def _ref_tap(x_ref):
    G, U, _, W2, D = x_ref.shape
    V = W2 // 2
    M = G * U * V
    planes = [x_ref[:, :, p] for p in (0, 1)]

    merged = [planes[p].reshape(M, 2 * D) for p in (0, 1)]

    def xfn(p, q):
        return merged[p][:, q * D:(q + 1) * D]

    return xfn


def _val_tap(y, U, V, D):
    merged = [y.reshape(U, 2, 2 * V, D)[:, p].reshape(U * V, 2 * D)
              for p in (0, 1)]

    def xfn(p, q):
        return merged[p][:, q * D:(q + 1) * D]

    return xfn


def _conv1_body(x_ref, w_ref, sb_ref, o_ref):
    o_ref[...] = _taps(_ref_tap(x_ref), w_ref, sb_ref)


def _conv1_call(z5, wst, sb, nsteps):
    N, U, _, W2, D = z5.shape
    V = W2 // 2
    g = N // nsteps
    mb = g * U * V
    return pl.pallas_call(
        _conv1_body,
        grid=(nsteps,),
        in_specs=[
            pl.BlockSpec((g, U, 2, W2, D), lambda j: (j, 0, 0, 0, 0)),
            pl.BlockSpec((4, D, D), lambda j: (0, 0, 0)),
            pl.BlockSpec((2, D), lambda j: (0, 0)),
        ],
        out_specs=pl.BlockSpec((mb, D), lambda j: (j, 0)),
        out_shape=jax.ShapeDtypeStruct((N * U * V, D), jnp.float32),
    )(z5, wst, sb)


def _conv2_body(x_ref, w_ref, sb_ref, o_ref):
    y = _taps(_ref_tap(x_ref), w_ref, sb_ref)
    D = y.shape[-1]
    y5 = y.reshape(2, 8, 8, 8, D)
    o_ref[...] = jnp.transpose(y5, (0, 2, 1, 3, 4)).reshape(16, 64, D)


def _conv2_call(z5, wst, sb):
    N, U, _, W2, D = z5.shape
    return pl.pallas_call(
        _conv2_body,
        grid=(4,),
        in_specs=[
            pl.BlockSpec((16, U, 2, W2, D), lambda j: (j, 0, 0, 0, 0)),
            pl.BlockSpec((4, D, D), lambda j: (0, 0, 0)),
            pl.BlockSpec((2, D), lambda j: (0, 0)),
        ],
        out_specs=pl.BlockSpec((16, 64, D), lambda j: (j, 0, 0)),
        out_shape=jax.ShapeDtypeStruct((64, 64, D), jnp.float32),
    )(z5, wst, sb)


def _tail_body(m_ref, w3_ref, sb3_ref, w4_ref, sb4_ref, w5_ref, sb5_ref,
               fw_ref, fb_ref, o_ref):
    D = m_ref.shape[-1]
    y3 = _taps(_ref_tap(m_ref), w3_ref, sb3_ref)
    y4 = _taps(_val_tap(y3, 16, 16, D), w4_ref, sb4_ref)
    y5 = _taps(_val_tap(y4, 8, 8, D), w5_ref, sb5_ref)
    h = jnp.sum(y5 * fw_ref[...], axis=1, keepdims=True) + fb_ref[0:1, 0:1]
    o_ref[...] = h


def _tail_call(m3, w3, sb3, w4, sb4, w5, sb5, fw, fb):
    return pl.pallas_call(
        _tail_body,
        out_shape=jax.ShapeDtypeStruct((64, 1), jnp.float32),
    )(m3, w3, sb3, w4, sb4, w5, sb5, fw, fb)


def _make_gate_body(E):
    def _gate_body(h_ref, c_ref, s_ref, wg_ref, o_ref):
        h = h_ref[...]
        re = jnp.dot(h, c_ref[...], preferred_element_type=jnp.float32)
        im = jnp.dot(h, s_ref[...], preferred_element_type=jnp.float32)
        amp = jnp.sqrt(re * re + im * im)
        lg = jnp.dot(amp, wg_ref[...], preferred_element_type=jnp.float32)
        col = lax.broadcasted_iota(jnp.int32, lg.shape, 1)
        o_ref[...] = jnp.where(col < E, lg, np.float32(-1e30))

    return _gate_body


def _logits_call(h4, cm, sm, wgp, E):
    B = h4.shape[0]
    return pl.pallas_call(
        _make_gate_body(E),
        out_shape=jax.ShapeDtypeStruct((B, 128), jnp.float32),
    )(h4, cm, sm, wgp)


def _sc_gate(logits_p):
    B = logits_p.shape[0]
    mesh = plsc.VectorSubcoreMesh(core_axis_name="c", subcore_axis_name="s")

    @functools.partial(
        pl.kernel,
        out_type=jax.ShapeDtypeStruct((B, 16), jnp.float32),
        mesh=mesh,
        compiler_params=pltpu.CompilerParams(needs_layout_passes=False),
        scratch_types=[
            pltpu.VMEM((16,), jnp.float32),
            pltpu.VMEM((16,), jnp.float32),
            pltpu.VMEM((16,), jnp.float32),
        ],
    )
    def k(lg_hbm, out_hbm, lvec, svec, gvec):
        wid = lax.axis_index("s") * 2 + lax.axis_index("c")

        @pl.when(wid < B)
        def _():
            pltpu.sync_copy(lg_hbm.at[wid, pl.ds(0, 16)], lvec)
            keys = lvec[...]
            lanes = lax.iota(jnp.int32, 16)
            sk, sv = plsc.sort_key_val(keys, lanes, descending=True)
            svec[...] = sk
            top1 = plsc.load_gather(svec, [jnp.zeros((16,), jnp.int32)])
            top2 = plsc.load_gather(svec, [jnp.full((16,), 1, jnp.int32)])
            e = jnp.exp(top2 - top1)
            one = jnp.full((16,), 1.0, jnp.float32)
            den = one + e
            g1 = one / den
            g2 = e / den
            vals = jnp.where(lanes == 0, g1, g2)
            gvec[...] = jnp.zeros((16,), jnp.float32)
            plsc.store_scatter(gvec, [sv], vals, mask=lanes < 2)
            pltpu.sync_copy(gvec, out_hbm.at[wid])

    return k(logits_p)


def kernel(x, conv_ws, conv_bs, bn_gammas, bn_betas, fuse_w, fuse_b, w_gate,
           training):
    B, T, H, W, D = x.shape
    N = B * T
    nf, E = w_gate.shape

    wsts, sbs = [], []
    for cw, cb, g, bb in zip(conv_ws, conv_bs, bn_gammas, bn_betas):
        wsts.append(jnp.transpose(cw, (2, 3, 0, 1)).reshape(4, D, D))
        sbs.append(jnp.stack([g, cb * g + bb]))

    z1 = x.reshape(N, H // 2, 2, W, D)
    y1 = _conv1_call(z1, wsts[0], sbs[0], 16)
    z2 = y1.reshape(N, 8, 2, 16, D)
    m3 = _conv2_call(z2, wsts[1], sbs[1])
    h = _tail_call(
        m3.reshape(1, 32, 2, 64, D), wsts[2], sbs[2], wsts[3], sbs[3],
        wsts[4], sbs[4], fuse_w.reshape(1, D), fuse_b.reshape(1, 1),
    )
    h4 = h.reshape(B, T)

    t = np.arange(T)[:, None].astype(np.float64)
    kk = np.arange(1, nf + 1)[None, :].astype(np.float64)
    ang = 2.0 * np.pi * t * kk / T
    cm = jnp.asarray(np.cos(ang) / np.sqrt(T), jnp.float32)
    sm = jnp.asarray(-np.sin(ang) / np.sqrt(T), jnp.float32)
    wgp = jnp.zeros((nf, 128), jnp.float32).at[:, :E].set(w_gate)

    lgp = _logits_call(h4, cm, sm, wgp, E)
    gates16 = _sc_gate(lgp)
    return gates16[:, :E]


# speedup vs baseline: 1.1953x; 1.0206x over previous
import functools

import numpy as np
import jax
import jax.numpy as jnp
from jax import lax
from jax.experimental import pallas as pl
from jax.experimental.pallas import tpu as pltpu
from jax.experimental.pallas import tpu_sc as plsc

_DN_T = (((1,), (1,)), ((), ()))


def _taps(xfn, w_ref, sb_ref):
    acc = None
    for p in (0, 1):
        for q in (0, 1):
            part = lax.dot_general(xfn(p, q), w_ref[2 * p + q], _DN_T,
                                   preferred_element_type=jnp.float32)
            acc = part if acc is None else acc + part
    acc = acc * sb_ref[0:1, :] + sb_ref[1:2, :]
    return jnp.where(acc >= 0.0, acc, np.float32(0.2) * acc)


def _ref_tap(x_ref):
    G, U, _, W2, D = x_ref.shape
    V = W2 // 2
    M = G * U * V
    planes = [x_ref[:, :, p] for p in (0, 1)]

    merged = [planes[p].reshape(M, 2 * D) for p in (0, 1)]

    def xfn(p, q):
        return merged[p][:, q * D:(q + 1) * D]

    return xfn


def _val_tap(y, U, V, D):
    merged = [y.reshape(U, 2, 2 * V, D)[:, p].reshape(U * V, 2 * D)
              for p in (0, 1)]

    def xfn(p, q):
        return merged[p][:, q * D:(q + 1) * D]

    return xfn


def _conv1_body(x_ref, w_ref, sb_ref, o_ref):
    o_ref[...] = _taps(_ref_tap(x_ref), w_ref, sb_ref)


def _conv1_call(z5, wst, sb, nsteps):
    N, U, _, W2, D = z5.shape
    V = W2 // 2
    g = N // nsteps
    mb = g * U * V
    return pl.pallas_call(
        _conv1_body,
        grid=(nsteps,),
        in_specs=[
            pl.BlockSpec((g, U, 2, W2, D), lambda j: (j, 0, 0, 0, 0)),
            pl.BlockSpec((4, D, D), lambda j: (0, 0, 0)),
            pl.BlockSpec((2, D), lambda j: (0, 0)),
        ],
        out_specs=pl.BlockSpec((mb, D), lambda j: (j, 0)),
        out_shape=jax.ShapeDtypeStruct((N * U * V, D), jnp.float32),
    )(z5, wst, sb)


def _conv2_body(x_ref, w_ref, sb_ref, o_ref):
    y = _taps(_ref_tap(x_ref), w_ref, sb_ref)
    D = y.shape[-1]
    y5 = y.reshape(2, 8, 8, 8, D)
    o_ref[...] = jnp.transpose(y5, (0, 2, 1, 3, 4)).reshape(16, 64, D)


def _conv2_call(z5, wst, sb):
    N, U, _, W2, D = z5.shape
    return pl.pallas_call(
        _conv2_body,
        grid=(4,),
        in_specs=[
            pl.BlockSpec((16, U, 2, W2, D), lambda j: (j, 0, 0, 0, 0)),
            pl.BlockSpec((4, D, D), lambda j: (0, 0, 0)),
            pl.BlockSpec((2, D), lambda j: (0, 0)),
        ],
        out_specs=pl.BlockSpec((16, 64, D), lambda j: (j, 0, 0)),
        out_shape=jax.ShapeDtypeStruct((64, 64, D), jnp.float32),
    )(z5, wst, sb)


def _make_tail_body(E):
    def _tail_body(m_ref, w3_ref, sb3_ref, w4_ref, sb4_ref, w5_ref, sb5_ref,
                   fw_ref, fb_ref, c_ref, s_ref, wg_ref, o_ref):
        D = m_ref.shape[-1]
        y3 = _taps(_ref_tap(m_ref), w3_ref, sb3_ref)
        y4 = _taps(_val_tap(y3, 16, 16, D), w4_ref, sb4_ref)
        y5 = _taps(_val_tap(y4, 8, 8, D), w5_ref, sb5_ref)
        h = (jnp.sum(y5 * fw_ref[...], axis=1, keepdims=True)
             + fb_ref[0:1, 0:1])
        h4 = h.reshape(4, 16)
        re = jnp.dot(h4, c_ref[...], preferred_element_type=jnp.float32)
        im = jnp.dot(h4, s_ref[...], preferred_element_type=jnp.float32)
        amp = jnp.sqrt(re * re + im * im)
        lg = jnp.dot(amp, wg_ref[...], preferred_element_type=jnp.float32)
        col = lax.broadcasted_iota(jnp.int32, lg.shape, 1)
        o_ref[...] = jnp.where(col < E, lg, np.float32(-1e30))

    return _tail_body


def _tail_call(m3, w3, sb3, w4, sb4, w5, sb5, fw, fb, cm, sm, wgp, E):
    return pl.pallas_call(
        _make_tail_body(E),
        out_shape=jax.ShapeDtypeStruct((4, 128), jnp.float32),
    )(m3, w3, sb3, w4, sb4, w5, sb5, fw, fb, cm, sm, wgp)


def _make_gate_body(E):
    def _gate_body(h_ref, c_ref, s_ref, wg_ref, o_ref):
        h = h_ref[...]
        re = jnp.dot(h, c_ref[...], preferred_element_type=jnp.float32)
        im = jnp.dot(h, s_ref[...], preferred_element_type=jnp.float32)
        amp = jnp.sqrt(re * re + im * im)
        lg = jnp.dot(amp, wg_ref[...], preferred_element_type=jnp.float32)
        col = lax.broadcasted_iota(jnp.int32, lg.shape, 1)
        o_ref[...] = jnp.where(col < E, lg, np.float32(-1e30))

    return _gate_body


def _logits_call(h4, cm, sm, wgp, E):
    B = h4.shape[0]
    return pl.pallas_call(
        _make_gate_body(E),
        out_shape=jax.ShapeDtypeStruct((B, 128), jnp.float32),
    )(h4, cm, sm, wgp)


def _sc_gate(logits_p):
    B = logits_p.shape[0]
    mesh = plsc.VectorSubcoreMesh(core_axis_name="c", subcore_axis_name="s")

    @functools.partial(
        pl.kernel,
        out_type=jax.ShapeDtypeStruct((B, 16), jnp.float32),
        mesh=mesh,
        compiler_params=pltpu.CompilerParams(needs_layout_passes=False),
        scratch_types=[
            pltpu.VMEM((16,), jnp.float32),
            pltpu.VMEM((16,), jnp.float32),
            pltpu.VMEM((16,), jnp.float32),
        ],
    )
    def k(lg_hbm, out_hbm, lvec, svec, gvec):
        wid = lax.axis_index("s") * 2 + lax.axis_index("c")

        @pl.when(wid < B)
        def _():
            pltpu.sync_copy(lg_hbm.at[wid, pl.ds(0, 16)], lvec)
            keys = lvec[...]
            lanes = lax.iota(jnp.int32, 16)
            sk, sv = plsc.sort_key_val(keys, lanes, descending=True)
            svec[...] = sk
            top1 = plsc.load_gather(svec, [jnp.zeros((16,), jnp.int32)])
            top2 = plsc.load_gather(svec, [jnp.full((16,), 1, jnp.int32)])
            e = jnp.exp(top2 - top1)
            one = jnp.full((16,), 1.0, jnp.float32)
            den = one + e
            g1 = one / den
            g2 = e / den
            vals = jnp.where(lanes == 0, g1, g2)
            gvec[...] = jnp.zeros((16,), jnp.float32)
            plsc.store_scatter(gvec, [sv], vals, mask=lanes < 2)
            pltpu.sync_copy(gvec, out_hbm.at[wid])

    return k(logits_p)


def kernel(x, conv_ws, conv_bs, bn_gammas, bn_betas, fuse_w, fuse_b, w_gate,
           training):
    B, T, H, W, D = x.shape
    N = B * T
    nf, E = w_gate.shape

    wsts, sbs = [], []
    for cw, cb, g, bb in zip(conv_ws, conv_bs, bn_gammas, bn_betas):
        wsts.append(jnp.transpose(cw, (2, 3, 0, 1)).reshape(4, D, D))
        sbs.append(jnp.stack([g, cb * g + bb]))

    z1 = x.reshape(N, H // 2, 2, W, D)
    y1 = _conv1_call(z1, wsts[0], sbs[0], 16)
    z2 = y1.reshape(N, 8, 2, 16, D)
    m3 = _conv2_call(z2, wsts[1], sbs[1])
    t = np.arange(T)[:, None].astype(np.float64)
    kk = np.arange(1, nf + 1)[None, :].astype(np.float64)
    ang = 2.0 * np.pi * t * kk / T
    cm = jnp.asarray(np.cos(ang) / np.sqrt(T), jnp.float32)
    sm = jnp.asarray(-np.sin(ang) / np.sqrt(T), jnp.float32)
    wgp = jnp.zeros((nf, 128), jnp.float32).at[:, :E].set(w_gate)

    lgp = _tail_call(
        m3.reshape(1, 32, 2, 64, D), wsts[2], sbs[2], wsts[3], sbs[3],
        wsts[4], sbs[4], fuse_w.reshape(1, D), fuse_b.reshape(1, 1),
        cm, sm, wgp, E,
    )
    gates16 = _sc_gate(lgp)
    return gates16[:, :E]
